# baseline (speedup 1.0000x reference)
import functools
import math

import jax
import jax.numpy as jnp
import numpy as np
from jax import lax
from jax.experimental import pallas as pl
from jax.experimental.pallas import tpu as pltpu
from jax.experimental.pallas import tpu_sc as plsc

VOCAB = 1000000
HIDDEN = 64
MAX_LEN = 512
BATCH = 4096
SEQ = 200

NC = 2
NS = 16
NW = NC * NS
LB = SEQ // 8
LANES = 16
BGROUPS = 128 // LANES


def _make_pe_np(hidden_size=HIDDEN, max_len=MAX_LEN):
    position = np.arange(0, max_len, dtype=np.float32)[:, None]
    div_term = np.exp(
        np.arange(0, hidden_size, 2, dtype=np.float32)
        * (-math.log(10000.0) / hidden_size)
    )
    pe = np.zeros((max_len, hidden_size), dtype=np.float32)
    pe[:, 0::2] = np.sin(position * div_term)
    pe[:, 1::2] = np.cos(position * div_term)
    return pe


_PE = _make_pe_np()[:SEQ].reshape(-1)


def _sc_embed(ids_p, table, pe):
    mesh = plsc.VectorSubcoreMesh(core_axis_name="c", subcore_axis_name="s")

    @functools.partial(
        pl.kernel,
        out_type=jax.ShapeDtypeStruct((SEQ, 8, NW, 8, 128), jnp.float32),
        mesh=mesh,
        compiler_params=pltpu.CompilerParams(
            use_tc_tiling_on_sc=False, needs_layout_passes=False
        ),
        scratch_types=(
            [pltpu.VMEM((SEQ * HIDDEN,), jnp.float32)]
            + [pltpu.VMEM((8, 128), jnp.int32)] * 2
            + [pltpu.VMEM((128, HIDDEN), jnp.float32)] * 2
            + [pltpu.VMEM((8, 8, 128), jnp.float32)] * 2
            + [pltpu.SemaphoreType.DMA] * 6
        ),
    )
    def k(ids_hbm, table_hbm, pe_hbm, out_hbm, pe_v, *rest):
        ibuf = rest[0:2]
        gbuf = rest[2:4]
        obuf = rest[4:6]
        isem = rest[6:8]
        gsem = rest[8:10]
        osem = rest[10:12]
        wid = lax.axis_index("s") * NC + lax.axis_index("c")

        pltpu.sync_copy(pe_hbm, pe_v)
        pltpu.sync_copy(ids_hbm.at[0, wid], ibuf[0])
        pltpu.async_copy(ids_hbm.at[1, wid], ibuf[1], isem[1])
        pltpu.async_copy(table_hbm.at[ibuf[0].at[0]], gbuf[0], gsem[0])

        rowg = [lax.iota(jnp.int32, LANES) + g * LANES for g in range(BGROUPS)]

        def pair(p, carry):
            for q in range(2):
                lb = p * 2 + q

                @pl.when(lb < LB)
                def _():
                    for ll in range(8):
                        l = lb * 8 + ll
                        g2 = ll % 2

                        if ll < 7:
                            nring, nrow = q, ll + 1
                        else:
                            nring, nrow = 1 - q, 0
                        if ll == 7:
                            @pl.when(lb < LB - 1)
                            def _():
                                pltpu.make_async_copy(
                                    ids_hbm.at[0, wid], ibuf[nring], isem[nring]
                                ).wait()

                        @pl.when(l < SEQ - 1)
                        def _():
                            pltpu.async_copy(
                                table_hbm.at[ibuf[nring].at[nrow]],
                                gbuf[1 - g2],
                                gsem[1 - g2],
                            )

                        pltpu.make_async_copy(
                            table_hbm.at[ibuf[q].at[ll]], gbuf[g2], gsem[g2]
                        ).wait()

                        if ll == 7:
                            @pl.when(lb < LB - 2)
                            def _():
                                pltpu.async_copy(
                                    ids_hbm.at[lb + 2, wid], ibuf[q], isem[q]
                                )

                        @pl.when(l >= 2)
                        def _():
                            pltpu.make_async_copy(
                                obuf[g2], out_hbm.at[0, :, wid], osem[g2]
                            ).wait()

                        @plsc.parallel_loop(0, HIDDEN, unroll=4)
                        def col(c):
                            colv = jnp.broadcast_to(c, (LANES,))
                            pev = plsc.load_gather(
                                pe_v, [jnp.broadcast_to(l * HIDDEN + c, (LANES,))]
                            )
                            for g in range(BGROUPS):
                                val = plsc.load_gather(gbuf[g2], [rowg[g], colv])
                                obuf[g2][c // 8, c % 8, pl.ds(g * LANES, LANES)] = (
                                    val + pev
                                )

                        pltpu.async_copy(
                            obuf[g2], out_hbm.at[l, :, wid], osem[g2]
                        )
            return carry

        lax.fori_loop(0, (LB + 1) // 2, pair, 0)
        for g2 in range(2):
            pltpu.make_async_copy(
                obuf[g2], out_hbm.at[0, :, wid], osem[g2]
            ).wait()

    return k(ids_p, table, pe)


def kernel(input_ids, table):
    ids_p = jnp.transpose(
        input_ids.astype(jnp.int32).reshape(NW, 128, LB, 8), (2, 0, 3, 1)
    )
    out5 = _sc_embed(ids_p, table, jnp.asarray(_PE))
    return jnp.transpose(out5, (2, 4, 0, 1, 3)).reshape(BATCH, SEQ, HIDDEN)
